# baseline (speedup 1.0000x reference)
import functools
import jax
import jax.numpy as jnp
from jax import lax
from jax.experimental import pallas as pl
from jax.experimental.pallas import tpu as pltpu
from jax.experimental.pallas import tpu_sc as plsc

_MODEL_DIM = 64
_BATCH = 4096
_SEQ = 200
_B_TOTAL = _BATCH * _SEQ

_info = plsc.get_sparse_core_info()
_NC = _info.num_cores
_NS = _info.num_subcores
_NW = _NC * _NS
_B_PER_W = _B_TOTAL // _NW
_CHUNK = 128
_N_CHUNKS = _B_PER_W // _CHUNK

_mesh = plsc.VectorSubcoreMesh(core_axis_name="c", subcore_axis_name="s")


@functools.partial(
    pl.kernel,
    mesh=_mesh,
    out_type=jax.ShapeDtypeStruct((_B_TOTAL, _MODEL_DIM), jnp.float32),
    scratch_types=[
        pltpu.VMEM((_N_CHUNKS, _CHUNK), jnp.int32),
        pltpu.VMEM((_CHUNK, _MODEL_DIM), jnp.float32),
        pltpu.SemaphoreType.DMA,
    ],
    compiler_params=pltpu.CompilerParams(use_tc_tiling_on_sc=False),
)
def _emb_lookup(idx_hbm, table_hbm, out_hbm, idx_v, rows_v, sem):
    wid = lax.axis_index("s") * _NC + lax.axis_index("c")
    base = wid * _B_PER_W
    pltpu.sync_copy(idx_hbm.at[wid], idx_v)

    def chunk_body(ci, carry):
        pltpu.async_copy(table_hbm.at[idx_v.at[ci]], rows_v, sem).wait()

        def scale_body(i, c):
            for j in range(_MODEL_DIM // 16):
                rows_v[i, pl.ds(j * 16, 16)] = (
                    rows_v[i, pl.ds(j * 16, 16)] * 8.0
                )
            return c

        lax.fori_loop(0, _CHUNK, scale_body, 0, unroll=4)
        pltpu.sync_copy(
            rows_v, out_hbm.at[pl.ds(base + ci * _CHUNK, _CHUNK)]
        )
        return carry

    lax.fori_loop(0, _N_CHUNKS, chunk_body, 0)


def kernel(token_ids_batch, embeddings_table):
    idx = token_ids_batch.reshape(_NW, _N_CHUNKS, _CHUNK).astype(jnp.int32)
    out = _emb_lookup(idx, embeddings_table)
    return out.reshape(_BATCH, _SEQ, _MODEL_DIM)


# speedup vs baseline: 1.0848x; 1.0848x over previous
import functools
import jax
import jax.numpy as jnp
from jax import lax
from jax.experimental import pallas as pl
from jax.experimental.pallas import tpu as pltpu
from jax.experimental.pallas import tpu_sc as plsc

_MODEL_DIM = 64
_BATCH = 4096
_SEQ = 200
_B_TOTAL = _BATCH * _SEQ

_info = plsc.get_sparse_core_info()
_NC = _info.num_cores
_NS = _info.num_subcores
_NW = _NC * _NS
_B_PER_W = _B_TOTAL // _NW
_CHUNK = 128
_N_CHUNKS = _B_PER_W // _CHUNK

_mesh = plsc.VectorSubcoreMesh(core_axis_name="c", subcore_axis_name="s")


@functools.partial(
    pl.kernel,
    mesh=_mesh,
    out_type=jax.ShapeDtypeStruct((_B_TOTAL, _MODEL_DIM), jnp.float32),
    scratch_types=[
        pltpu.VMEM((_N_CHUNKS, _CHUNK), jnp.int32),
        pltpu.VMEM((_CHUNK, _MODEL_DIM), jnp.float32),
        pltpu.VMEM((_CHUNK, _MODEL_DIM), jnp.float32),
        pltpu.SemaphoreType.DMA,
        pltpu.SemaphoreType.DMA,
        pltpu.SemaphoreType.DMA,
        pltpu.SemaphoreType.DMA,
    ],
    compiler_params=pltpu.CompilerParams(use_tc_tiling_on_sc=False),
)
def _emb_lookup(
    idx_hbm, table_hbm, out_hbm,
    idx_v, rows0, rows1, gsem0, gsem1, osem0, osem1,
):
    rows = [rows0, rows1]
    gsem = [gsem0, gsem1]
    osem = [osem0, osem1]
    wid = lax.axis_index("s") * _NC + lax.axis_index("c")
    base = wid * _B_PER_W
    pltpu.sync_copy(idx_hbm.at[wid], idx_v)
    pltpu.async_copy(table_hbm.at[idx_v.at[0]], rows0, gsem0)

    def outer(g, carry):
        for b in range(2):
            ci = 2 * g + b
            nb = 1 - b
            pltpu.make_async_copy(
                table_hbm.at[idx_v.at[ci]], rows[b], gsem[b]
            ).wait()

            @pl.when(jnp.logical_and(ci >= 1, ci + 1 < _N_CHUNKS))
            def _():
                pltpu.make_async_copy(
                    rows[nb],
                    out_hbm.at[pl.ds(base + (ci - 1) * _CHUNK, _CHUNK)],
                    osem[nb],
                ).wait()
                pltpu.async_copy(
                    table_hbm.at[idx_v.at[ci + 1]], rows[nb], gsem[nb]
                )

            @pl.when(ci == 0)
            def _():
                pltpu.async_copy(
                    table_hbm.at[idx_v.at[1]], rows[nb], gsem[nb]
                )

            def scale_body(i, c):
                for j in range(_MODEL_DIM // 16):
                    rows[b][i, pl.ds(j * 16, 16)] = (
                        rows[b][i, pl.ds(j * 16, 16)] * 8.0
                    )
                return c

            lax.fori_loop(0, _CHUNK, scale_body, 0, unroll=4)
            pltpu.async_copy(
                rows[b],
                out_hbm.at[pl.ds(base + ci * _CHUNK, _CHUNK)],
                osem[b],
            )
        return carry

    lax.fori_loop(0, _N_CHUNKS // 2, outer, 0)
    pltpu.make_async_copy(
        rows0, out_hbm.at[pl.ds(base + (_N_CHUNKS - 2) * _CHUNK, _CHUNK)], osem0
    ).wait()
    pltpu.make_async_copy(
        rows1, out_hbm.at[pl.ds(base + (_N_CHUNKS - 1) * _CHUNK, _CHUNK)], osem1
    ).wait()


def kernel(token_ids_batch, embeddings_table):
    idx = token_ids_batch.reshape(_NW, _N_CHUNKS, _CHUNK).astype(jnp.int32)
    out = _emb_lookup(idx, embeddings_table)
    return out.reshape(_BATCH, _SEQ, _MODEL_DIM)
